# baseline (speedup 1.0000x reference)
import jax
import jax.numpy as jnp
from jax.experimental import pallas as pl

N_AGENTS = 100
BATCH = 100
EQU = 2
INV = 6
HID = 64
N_LAYERS = 2


def _silu(v):
    return v * (0.5 * jnp.tanh(0.5 * v) + 0.5)


def _dot3(a, w):
    return jax.lax.dot_general(a, w, (((2,), (0,)), ((), ())),
                               preferred_element_type=jnp.float32)


def _egnn_kernel(x0c_ref, x1c_ref, hin_ref,
                 W_emb_ref, b_emb_ref,
                 eW1_ref, eb1_ref, eW2_ref, eb2_ref,
                 nW1_ref, nb1_ref, nW2_ref, nb2_ref,
                 cW1_ref, cb1_ref, cW2rep_ref,
                 fc1_ref, fc1b_ref, fc2_ref, fc2b_ref,
                 out_ref):
    n = N_AGENTS
    x0 = jnp.broadcast_to(x0c_ref[0], (n, HID))
    x1 = jnp.broadcast_to(x1c_ref[0], (n, HID))
    hin = hin_ref[0]

    h = jnp.dot(hin, W_emb_ref[...], preferred_element_type=jnp.float32) \
        + b_emb_ref[...]

    ii3 = jax.lax.broadcasted_iota(jnp.int32, (n, n, HID), 0)
    jj3 = jax.lax.broadcasted_iota(jnp.int32, (n, n, HID), 1)
    dmask3 = (ii3 != jj3).astype(jnp.float32)

    for i in range(N_LAYERS):
        d0 = x0[:, None, :] - x0[None, :, :]
        d1 = x1[:, None, :] - x1[None, :, :]
        radial = d0 * d0 + d1 * d1
        rn = 1.0 / (jnp.sqrt(radial) + 1e-8)
        nd0 = d0 * rn
        nd1 = d1 * rn

        A = jnp.dot(h, eW1_ref[i, :HID, :],
                    preferred_element_type=jnp.float32) + eb1_ref[i]
        B = jnp.dot(h, eW1_ref[i, HID:2 * HID, :],
                    preferred_element_type=jnp.float32)
        wr = eW1_ref[i, 2 * HID:2 * HID + 1, :]
        e1 = A[:, None, :] + B[None, :, :] + radial * wr[None]
        m = _silu(e1)
        m = _silu(_dot3(m, eW2_ref[i]) + eb2_ref[i][None])

        ch = _silu(_dot3(m, cW1_ref[i]) + cb1_ref[i][None])
        cm = jnp.tanh(_dot3(ch, cW2rep_ref[i]))

        x0 = x0 + jnp.sum(nd0 * cm, axis=1) * (1.0 / 99.0)
        x1 = x1 + jnp.sum(nd1 * cm, axis=1) * (1.0 / 99.0)

        hagg = jnp.sum(m * dmask3, axis=1)
        n1 = (jnp.dot(h, nW1_ref[i, :HID, :],
                      preferred_element_type=jnp.float32)
              + jnp.dot(hagg, nW1_ref[i, HID:, :],
                        preferred_element_type=jnp.float32)
              + nb1_ref[i])
        out = jnp.dot(_silu(n1), nW2_ref[i],
                      preferred_element_type=jnp.float32) + nb2_ref[i]
        h = h + out

    xs = x0 * x0 + x1 * x1
    z = jnp.tanh(xs * fc1_ref[0:1, :]
                 + jnp.dot(h, fc1_ref[1:, :],
                           preferred_element_type=jnp.float32)
                 + fc1b_ref[...])
    v = jnp.dot(z, fc2_ref[...],
                preferred_element_type=jnp.float32) + fc2b_ref[...]
    out_ref[0] = jnp.sum(v, axis=0, keepdims=True) * (1.0 / N_AGENTS)


def kernel(cent_obs, rnn_states, masks, edge_index, W_emb, b_emb,
           eW1, eb1, eW2, eb2, nW1, nb1, nW2, nb2, cW1, cb1, cW2,
           fc1_W, fc1_b, fc2_W, fc2_b):
    del masks, edge_index
    co = cent_obs.reshape(BATCH, N_AGENTS, EQU + INV)
    x0c = co[:, :, 0:1]
    x1c = co[:, :, 1:2]
    hin = co[:, :, EQU:]

    b_emb2 = b_emb.reshape(1, HID)
    eb1r = eb1.reshape(N_LAYERS, 1, HID)
    eb2r = eb2.reshape(N_LAYERS, 1, HID)
    nb1r = nb1.reshape(N_LAYERS, 1, HID)
    nb2r = nb2.reshape(N_LAYERS, 1, HID)
    cb1r = cb1.reshape(N_LAYERS, 1, HID)
    cW2rep = jnp.broadcast_to(cW2, (N_LAYERS, HID, HID))
    fc1b = fc1_b.reshape(1, HID)
    fc2b = fc2_b.reshape(1, 1)

    def bspec(shape):
        nd = len(shape)
        return pl.BlockSpec((1,) + shape[1:], lambda b: (b,) + (0,) * (nd - 1))

    def wspec(shape):
        nd = len(shape)
        return pl.BlockSpec(shape, lambda b: (0,) * nd)

    value = pl.pallas_call(
        _egnn_kernel,
        grid=(BATCH,),
        in_specs=[
            bspec(x0c.shape), bspec(x1c.shape), bspec(hin.shape),
            wspec(W_emb.shape), wspec(b_emb2.shape),
            wspec(eW1.shape), wspec(eb1r.shape),
            wspec(eW2.shape), wspec(eb2r.shape),
            wspec(nW1.shape), wspec(nb1r.shape),
            wspec(nW2.shape), wspec(nb2r.shape),
            wspec(cW1.shape), wspec(cb1r.shape), wspec(cW2rep.shape),
            wspec(fc1_W.shape), wspec(fc1b.shape),
            wspec(fc2_W.shape), wspec(fc2b.shape),
        ],
        out_specs=pl.BlockSpec((1, 1, 1), lambda b: (b, 0, 0)),
        out_shape=jax.ShapeDtypeStruct((BATCH, 1, 1), jnp.float32),
    )(x0c, x1c, hin, W_emb, b_emb2, eW1, eb1r, eW2, eb2r,
      nW1, nb1r, nW2, nb2r, cW1, cb1r, cW2rep, fc1_W, fc1b, fc2_W, fc2b)

    return (value.reshape(BATCH, 1), rnn_states)


# speedup vs baseline: 1.3389x; 1.3389x over previous
import jax
import jax.numpy as jnp
from jax.experimental import pallas as pl

N_AGENTS = 100
BATCH = 100
EQU = 2
INV = 6
HID = 64
N_LAYERS = 2


def _silu(v):
    return v * (0.5 * jnp.tanh(0.5 * v) + 0.5)


def _dot3(a, w):
    return jax.lax.dot_general(a, w, (((2,), (0,)), ((), ())),
                               preferred_element_type=jnp.float32)


def _egnn_kernel(x0c_ref, x1c_ref, hin_ref,
                 W_emb_ref, b_emb_ref,
                 eW1_ref, eb1_ref, eW2_ref, eb2_ref,
                 nW1_ref, nb1_ref, nW2_ref, nb2_ref,
                 cW1_ref, cb1_ref, cW2t_ref,
                 fc1_ref, fc1b_ref, fc2_ref, fc2b_ref,
                 out_ref):
    n = N_AGENTS
    x0c = x0c_ref[0]
    x1c = x1c_ref[0]
    x0r = x0c.reshape(1, n)
    x1r = x1c.reshape(1, n)
    hin = hin_ref[0]

    h = jnp.dot(hin, W_emb_ref[...], preferred_element_type=jnp.float32) \
        + b_emb_ref[...]

    ii3 = jax.lax.broadcasted_iota(jnp.int32, (n, n, HID), 0)
    jj3 = jax.lax.broadcasted_iota(jnp.int32, (n, n, HID), 1)
    dmask3 = (ii3 != jj3).astype(jnp.float32)

    for i in range(N_LAYERS):
        d0 = x0c - x0r
        d1 = x1c - x1r
        radial = d0 * d0 + d1 * d1
        rn = 1.0 / (jnp.sqrt(radial) + 1e-8)
        nd0 = d0 * rn
        nd1 = d1 * rn

        A = jnp.dot(h, eW1_ref[i, :HID, :],
                    preferred_element_type=jnp.float32) + eb1_ref[i]
        B = jnp.dot(h, eW1_ref[i, HID:2 * HID, :],
                    preferred_element_type=jnp.float32)
        wr = eW1_ref[i, 2 * HID:2 * HID + 1, :]
        xs2 = x0c * x0c + x1c * x1c
        A2 = A + xs2 * wr
        B2 = B + xs2 * wr
        G0 = x0c * wr * (-2.0)
        G1 = x1c * wr * (-2.0)
        e1 = (A2[:, None, :] + B2[None, :, :]
              + x0c[:, None, :] * G0[None, :, :]
              + x1c[:, None, :] * G1[None, :, :])
        m = _silu(e1)
        m = _silu(_dot3(m, eW2_ref[i]) + eb2_ref[i][None])

        ch = _silu(_dot3(m, cW1_ref[i]) + cb1_ref[i][None])
        cm = jnp.tanh(jnp.sum(ch * cW2t_ref[i][None], axis=2))

        x0c = x0c + jnp.sum(nd0 * cm, axis=1, keepdims=True) * (1.0 / 99.0)
        x1c = x1c + jnp.sum(nd1 * cm, axis=1, keepdims=True) * (1.0 / 99.0)
        x0r = x0c.reshape(1, n)
        x1r = x1c.reshape(1, n)

        hagg = jnp.sum(m * dmask3, axis=1)
        n1 = (jnp.dot(h, nW1_ref[i, :HID, :],
                      preferred_element_type=jnp.float32)
              + jnp.dot(hagg, nW1_ref[i, HID:, :],
                        preferred_element_type=jnp.float32)
              + nb1_ref[i])
        out = jnp.dot(_silu(n1), nW2_ref[i],
                      preferred_element_type=jnp.float32) + nb2_ref[i]
        h = h + out

    xs = x0c * x0c + x1c * x1c
    z = jnp.tanh(xs * fc1_ref[0:1, :]
                 + jnp.dot(h, fc1_ref[1:, :],
                           preferred_element_type=jnp.float32)
                 + fc1b_ref[...])
    v = jnp.dot(z, fc2_ref[...],
                preferred_element_type=jnp.float32) + fc2b_ref[...]
    out_ref[0] = jnp.sum(v, axis=0, keepdims=True) * (1.0 / N_AGENTS)


def kernel(cent_obs, rnn_states, masks, edge_index, W_emb, b_emb,
           eW1, eb1, eW2, eb2, nW1, nb1, nW2, nb2, cW1, cb1, cW2,
           fc1_W, fc1_b, fc2_W, fc2_b):
    del masks, edge_index
    co = cent_obs.reshape(BATCH, N_AGENTS, EQU + INV)
    x0c = co[:, :, 0:1]
    x1c = co[:, :, 1:2]
    hin = co[:, :, EQU:]

    b_emb2 = b_emb.reshape(1, HID)
    eb1r = eb1.reshape(N_LAYERS, 1, HID)
    eb2r = eb2.reshape(N_LAYERS, 1, HID)
    nb1r = nb1.reshape(N_LAYERS, 1, HID)
    nb2r = nb2.reshape(N_LAYERS, 1, HID)
    cb1r = cb1.reshape(N_LAYERS, 1, HID)
    cW2t = jnp.transpose(cW2, (0, 2, 1))
    fc1b = fc1_b.reshape(1, HID)
    fc2b = fc2_b.reshape(1, 1)

    def bspec(shape):
        nd = len(shape)
        return pl.BlockSpec((1,) + shape[1:], lambda b: (b,) + (0,) * (nd - 1))

    def wspec(shape):
        nd = len(shape)
        return pl.BlockSpec(shape, lambda b: (0,) * nd)

    value = pl.pallas_call(
        _egnn_kernel,
        grid=(BATCH,),
        in_specs=[
            bspec(x0c.shape), bspec(x1c.shape), bspec(hin.shape),
            wspec(W_emb.shape), wspec(b_emb2.shape),
            wspec(eW1.shape), wspec(eb1r.shape),
            wspec(eW2.shape), wspec(eb2r.shape),
            wspec(nW1.shape), wspec(nb1r.shape),
            wspec(nW2.shape), wspec(nb2r.shape),
            wspec(cW1.shape), wspec(cb1r.shape), wspec(cW2t.shape),
            wspec(fc1_W.shape), wspec(fc1b.shape),
            wspec(fc2_W.shape), wspec(fc2b.shape),
        ],
        out_specs=pl.BlockSpec((1, 1, 1), lambda b: (b, 0, 0)),
        out_shape=jax.ShapeDtypeStruct((BATCH, 1, 1), jnp.float32),
    )(x0c, x1c, hin, W_emb, b_emb2, eW1, eb1r, eW2, eb2r,
      nW1, nb1r, nW2, nb2r, cW1, cb1r, cW2t, fc1_W, fc1b, fc2_W, fc2b)

    return (value.reshape(BATCH, 1), rnn_states)


# speedup vs baseline: 1.7602x; 1.3146x over previous
import jax
import jax.numpy as jnp
from jax.experimental import pallas as pl

N_AGENTS = 100
BATCH = 100
EQU = 2
INV = 6
HID = 64
N_LAYERS = 2
PK = 2
PH = PK * HID


def _silu(v):
    return v * (0.5 * jnp.tanh(0.5 * v) + 0.5)


def _dot3(a, w):
    return jax.lax.dot_general(a, w, (((2,), (0,)), ((), ())),
                               preferred_element_type=jnp.float32)


def _egnn_kernel(xpi0_ref, xpi1_ref, hinp_ref,
                 W_embd_ref, b_embt_ref,
                 eW1a_ref, eW1b_ref, ewr_ref, eb1t_ref,
                 eW2d_ref, eb2t_ref,
                 nW1a_ref, nW1b_ref, nb1t_ref, nW2d_ref, nb2t_ref,
                 cW1d_ref, cb1t_ref, cW2t_ref,
                 fc1r0_ref, fc1d_ref, fc1bt_ref, fc2t_ref,
                 out_ref):
    n = N_AGENTS
    xp0 = xpi0_ref[0]
    xp1 = xpi1_ref[0]
    hinp = hinp_ref[0]

    h = jnp.dot(hinp, W_embd_ref[...], preferred_element_type=jnp.float32) \
        + b_embt_ref[...]

    ii3 = jax.lax.broadcasted_iota(jnp.int32, (n, n, PH), 0)
    jj3 = jax.lax.broadcasted_iota(jnp.int32, (n, n, PH), 1)
    dmask3 = (ii3 != jj3).astype(jnp.float32)

    def halves(xp):
        return xp[:, 0:1], xp[:, HID:HID + 1]

    for i in range(N_LAYERS):
        wr = ewr_ref[i]
        x0a, x0b = halves(xp0)
        x1a, x1b = halves(xp1)

        A = jnp.dot(h, eW1a_ref[i], preferred_element_type=jnp.float32) \
            + eb1t_ref[i]
        B = jnp.dot(h, eW1b_ref[i], preferred_element_type=jnp.float32)
        xs2 = xp0 * xp0 + xp1 * xp1
        A2 = A + xs2 * wr
        B2 = B + xs2 * wr
        G0 = xp0 * wr * (-2.0)
        G1 = xp1 * wr * (-2.0)
        e1 = (A2[:, None, :] + B2[None, :, :]
              + xp0[:, None, :] * G0[None, :, :]
              + xp1[:, None, :] * G1[None, :, :])
        m = _silu(e1)
        m = _silu(_dot3(m, eW2d_ref[i]) + eb2t_ref[i][None])

        ch = _silu(_dot3(m, cW1d_ref[i]) + cb1t_ref[i][None])
        cm_a = jnp.tanh(jnp.sum(ch[:, :, :HID] * cW2t_ref[i][None], axis=2))
        cm_b = jnp.tanh(jnp.sum(ch[:, :, HID:] * cW2t_ref[i][None], axis=2))

        def coord_agg(x0c, x1c, cm):
            x0r = x0c.reshape(1, n)
            x1r = x1c.reshape(1, n)
            d0 = x0c - x0r
            d1 = x1c - x1r
            rn = 1.0 / (jnp.sqrt(d0 * d0 + d1 * d1) + 1e-8)
            g = rn * cm
            a0 = jnp.sum(d0 * g, axis=1, keepdims=True) * (1.0 / 99.0)
            a1 = jnp.sum(d1 * g, axis=1, keepdims=True) * (1.0 / 99.0)
            return a0, a1

        a0a, a1a = coord_agg(x0a, x1a, cm_a)
        a0b, a1b = coord_agg(x0b, x1b, cm_b)
        xp0 = xp0 + jnp.concatenate(
            [jnp.broadcast_to(a0a, (n, HID)),
             jnp.broadcast_to(a0b, (n, HID))], axis=1)
        xp1 = xp1 + jnp.concatenate(
            [jnp.broadcast_to(a1a, (n, HID)),
             jnp.broadcast_to(a1b, (n, HID))], axis=1)

        hagg = jnp.sum(m * dmask3, axis=1)
        n1 = (jnp.dot(h, nW1a_ref[i], preferred_element_type=jnp.float32)
              + jnp.dot(hagg, nW1b_ref[i], preferred_element_type=jnp.float32)
              + nb1t_ref[i])
        out = jnp.dot(_silu(n1), nW2d_ref[i],
                      preferred_element_type=jnp.float32) + nb2t_ref[i]
        h = h + out

    xs = xp0 * xp0 + xp1 * xp1
    z = jnp.tanh(xs * fc1r0_ref[...]
                 + jnp.dot(h, fc1d_ref[...],
                           preferred_element_type=jnp.float32)
                 + fc1bt_ref[...])
    q = z * fc2t_ref[...]
    out_ref[0] = jnp.sum(q, axis=0, keepdims=True)


def _bd(w):
    return jnp.kron(jnp.eye(PK, dtype=w.dtype), w)


def _tile(v):
    return jnp.tile(v.reshape(1, -1), (1, PK))


def kernel(cent_obs, rnn_states, masks, edge_index, W_emb, b_emb,
           eW1, eb1, eW2, eb2, nW1, nb1, nW2, nb2, cW1, cb1, cW2,
           fc1_W, fc1_b, fc2_W, fc2_b):
    del masks, edge_index
    G = BATCH // PK
    co = cent_obs.reshape(G, PK, N_AGENTS, EQU + INV)
    xpi0 = jnp.repeat(jnp.transpose(co[:, :, :, 0], (0, 2, 1)), HID, axis=2)
    xpi1 = jnp.repeat(jnp.transpose(co[:, :, :, 1], (0, 2, 1)), HID, axis=2)
    hinp = jnp.transpose(co[:, :, :, EQU:], (0, 2, 1, 3)).reshape(
        G, N_AGENTS, PK * INV)

    W_embd = _bd(W_emb)
    b_embt = _tile(b_emb)
    eW1a = jnp.stack([_bd(eW1[i, :HID]) for i in range(N_LAYERS)])
    eW1b = jnp.stack([_bd(eW1[i, HID:2 * HID]) for i in range(N_LAYERS)])
    ewr = jnp.stack([_tile(eW1[i, 2 * HID]) for i in range(N_LAYERS)])
    eb1t = jnp.stack([_tile(eb1[i]) for i in range(N_LAYERS)])
    eW2d = jnp.stack([_bd(eW2[i]) for i in range(N_LAYERS)])
    eb2t = jnp.stack([_tile(eb2[i]) for i in range(N_LAYERS)])
    nW1a = jnp.stack([_bd(nW1[i, :HID]) for i in range(N_LAYERS)])
    nW1b = jnp.stack([_bd(nW1[i, HID:]) for i in range(N_LAYERS)])
    nb1t = jnp.stack([_tile(nb1[i]) for i in range(N_LAYERS)])
    nW2d = jnp.stack([_bd(nW2[i]) for i in range(N_LAYERS)])
    nb2t = jnp.stack([_tile(nb2[i]) for i in range(N_LAYERS)])
    cW1d = jnp.stack([_bd(cW1[i]) for i in range(N_LAYERS)])
    cb1t = jnp.stack([_tile(cb1[i]) for i in range(N_LAYERS)])
    cW2t = jnp.transpose(cW2, (0, 2, 1))
    fc1r0 = _tile(fc1_W[0])
    fc1d = _bd(fc1_W[1:])
    fc1bt = _tile(fc1_b)
    fc2t = _tile(fc2_W[:, 0])

    def bspec(shape):
        nd = len(shape)
        return pl.BlockSpec((1,) + shape[1:], lambda b: (b,) + (0,) * (nd - 1))

    def wspec(shape):
        nd = len(shape)
        return pl.BlockSpec(shape, lambda b: (0,) * nd)

    ins = [xpi0, xpi1, hinp, W_embd, b_embt,
           eW1a, eW1b, ewr, eb1t, eW2d, eb2t,
           nW1a, nW1b, nb1t, nW2d, nb2t,
           cW1d, cb1t, cW2t,
           fc1r0, fc1d, fc1bt, fc2t]
    specs = [bspec(xpi0.shape), bspec(xpi1.shape), bspec(hinp.shape)] + \
            [wspec(a.shape) for a in ins[3:]]

    sums = pl.pallas_call(
        _egnn_kernel,
        grid=(G,),
        in_specs=specs,
        out_specs=pl.BlockSpec((1, 1, PH), lambda b: (b, 0, 0)),
        out_shape=jax.ShapeDtypeStruct((G, 1, PH), jnp.float32),
    )(*ins)

    value = sums[:, 0, :].reshape(G * PK, HID).sum(axis=1) * (1.0 / N_AGENTS)
    value = value.reshape(BATCH, 1) + fc2_b.reshape(1, 1)
    return (value, rnn_states)


# speedup vs baseline: 1.8011x; 1.0232x over previous
import jax
import jax.numpy as jnp
from jax.experimental import pallas as pl

N_AGENTS = 100
BATCH = 100
EQU = 2
INV = 6
HID = 64
N_LAYERS = 2
PK = 2
PH = PK * HID


def _silu(v):
    return v * (0.5 * jnp.tanh(0.5 * v) + 0.5)


def _dot3(a, w):
    return jax.lax.dot_general(a, w, (((2,), (0,)), ((), ())),
                               preferred_element_type=jnp.float32)


def _egnn_kernel(xpi0_ref, xpi1_ref, hinp_ref,
                 W_embd_ref, b_embt_ref,
                 eW1a_ref, eW1b_ref, ewr_ref, eb1t_ref,
                 eW2d_ref, eb2t_ref,
                 nW1a_ref, nW1b_ref, nb1t_ref, nW2d_ref, nb2t_ref,
                 cW1d_ref, cb1t_ref, cW2t_ref,
                 fc1r0_ref, fc1d_ref, fc1bt_ref, fc2t_ref,
                 out_ref):
    n = N_AGENTS
    xp0 = xpi0_ref[0]
    xp1 = xpi1_ref[0]
    hinp = hinp_ref[0]

    h = jnp.dot(hinp, W_embd_ref[...], preferred_element_type=jnp.float32) \
        + b_embt_ref[...]

    def halves(xp):
        return xp[:, 0:1], xp[:, HID:HID + 1]

    for i in range(N_LAYERS):
        wr = ewr_ref[i]
        x0a, x0b = halves(xp0)
        x1a, x1b = halves(xp1)

        A = jnp.dot(h, eW1a_ref[i], preferred_element_type=jnp.float32) \
            + eb1t_ref[i]
        B = jnp.dot(h, eW1b_ref[i], preferred_element_type=jnp.float32)
        xs2 = xp0 * xp0 + xp1 * xp1
        A2 = A + xs2 * wr
        B2 = B + xs2 * wr
        G0 = xp0 * wr * (-2.0)
        G1 = xp1 * wr * (-2.0)
        e1 = (A2[:, None, :] + B2[None, :, :]
              + xp0[:, None, :] * G0[None, :, :]
              + xp1[:, None, :] * G1[None, :, :])
        m = _silu(e1)
        m = _silu(_dot3(m, eW2d_ref[i]) + eb2t_ref[i][None])

        ch = _silu(_dot3(m, cW1d_ref[i]) + cb1t_ref[i][None])
        cm_a = jnp.tanh(jnp.sum(ch[:, :, :HID] * cW2t_ref[i][None], axis=2))
        cm_b = jnp.tanh(jnp.sum(ch[:, :, HID:] * cW2t_ref[i][None], axis=2))

        def coord_agg(x0c, x1c, cm):
            x0r = x0c.reshape(1, n)
            x1r = x1c.reshape(1, n)
            d0 = x0c - x0r
            d1 = x1c - x1r
            rn = 1.0 / (jnp.sqrt(d0 * d0 + d1 * d1) + 1e-8)
            g = rn * cm
            a0 = jnp.sum(d0 * g, axis=1, keepdims=True) * (1.0 / 99.0)
            a1 = jnp.sum(d1 * g, axis=1, keepdims=True) * (1.0 / 99.0)
            return a0, a1

        a0a, a1a = coord_agg(x0a, x1a, cm_a)
        a0b, a1b = coord_agg(x0b, x1b, cm_b)
        xp0 = xp0 + jnp.concatenate(
            [jnp.broadcast_to(a0a, (n, HID)),
             jnp.broadcast_to(a0b, (n, HID))], axis=1)
        xp1 = xp1 + jnp.concatenate(
            [jnp.broadcast_to(a1a, (n, HID)),
             jnp.broadcast_to(a1b, (n, HID))], axis=1)

        ed = A2 + B2 + xp0 * G0 + xp1 * G1
        md = _silu(jnp.dot(_silu(ed), eW2d_ref[i],
                           preferred_element_type=jnp.float32) + eb2t_ref[i])
        hagg = jnp.sum(m, axis=1) - md
        n1 = (jnp.dot(h, nW1a_ref[i], preferred_element_type=jnp.float32)
              + jnp.dot(hagg, nW1b_ref[i], preferred_element_type=jnp.float32)
              + nb1t_ref[i])
        out = jnp.dot(_silu(n1), nW2d_ref[i],
                      preferred_element_type=jnp.float32) + nb2t_ref[i]
        h = h + out

    xs = xp0 * xp0 + xp1 * xp1
    z = jnp.tanh(xs * fc1r0_ref[...]
                 + jnp.dot(h, fc1d_ref[...],
                           preferred_element_type=jnp.float32)
                 + fc1bt_ref[...])
    q = z * fc2t_ref[...]
    out_ref[0] = jnp.sum(q, axis=0, keepdims=True)


def _bd(w):
    return jnp.kron(jnp.eye(PK, dtype=w.dtype), w)


def _tile(v):
    return jnp.tile(v.reshape(1, -1), (1, PK))


def kernel(cent_obs, rnn_states, masks, edge_index, W_emb, b_emb,
           eW1, eb1, eW2, eb2, nW1, nb1, nW2, nb2, cW1, cb1, cW2,
           fc1_W, fc1_b, fc2_W, fc2_b):
    del masks, edge_index
    G = BATCH // PK
    co = cent_obs.reshape(G, PK, N_AGENTS, EQU + INV)
    xpi0 = jnp.repeat(jnp.transpose(co[:, :, :, 0], (0, 2, 1)), HID, axis=2)
    xpi1 = jnp.repeat(jnp.transpose(co[:, :, :, 1], (0, 2, 1)), HID, axis=2)
    hinp = jnp.transpose(co[:, :, :, EQU:], (0, 2, 1, 3)).reshape(
        G, N_AGENTS, PK * INV)

    W_embd = _bd(W_emb)
    b_embt = _tile(b_emb)
    eW1a = jnp.stack([_bd(eW1[i, :HID]) for i in range(N_LAYERS)])
    eW1b = jnp.stack([_bd(eW1[i, HID:2 * HID]) for i in range(N_LAYERS)])
    ewr = jnp.stack([_tile(eW1[i, 2 * HID]) for i in range(N_LAYERS)])
    eb1t = jnp.stack([_tile(eb1[i]) for i in range(N_LAYERS)])
    eW2d = jnp.stack([_bd(eW2[i]) for i in range(N_LAYERS)])
    eb2t = jnp.stack([_tile(eb2[i]) for i in range(N_LAYERS)])
    nW1a = jnp.stack([_bd(nW1[i, :HID]) for i in range(N_LAYERS)])
    nW1b = jnp.stack([_bd(nW1[i, HID:]) for i in range(N_LAYERS)])
    nb1t = jnp.stack([_tile(nb1[i]) for i in range(N_LAYERS)])
    nW2d = jnp.stack([_bd(nW2[i]) for i in range(N_LAYERS)])
    nb2t = jnp.stack([_tile(nb2[i]) for i in range(N_LAYERS)])
    cW1d = jnp.stack([_bd(cW1[i]) for i in range(N_LAYERS)])
    cb1t = jnp.stack([_tile(cb1[i]) for i in range(N_LAYERS)])
    cW2t = jnp.transpose(cW2, (0, 2, 1))
    fc1r0 = _tile(fc1_W[0])
    fc1d = _bd(fc1_W[1:])
    fc1bt = _tile(fc1_b)
    fc2t = _tile(fc2_W[:, 0])

    def bspec(shape):
        nd = len(shape)
        return pl.BlockSpec((1,) + shape[1:], lambda b: (b,) + (0,) * (nd - 1))

    def wspec(shape):
        nd = len(shape)
        return pl.BlockSpec(shape, lambda b: (0,) * nd)

    ins = [xpi0, xpi1, hinp, W_embd, b_embt,
           eW1a, eW1b, ewr, eb1t, eW2d, eb2t,
           nW1a, nW1b, nb1t, nW2d, nb2t,
           cW1d, cb1t, cW2t,
           fc1r0, fc1d, fc1bt, fc2t]
    specs = [bspec(xpi0.shape), bspec(xpi1.shape), bspec(hinp.shape)] + \
            [wspec(a.shape) for a in ins[3:]]

    sums = pl.pallas_call(
        _egnn_kernel,
        grid=(G,),
        in_specs=specs,
        out_specs=pl.BlockSpec((1, 1, PH), lambda b: (b, 0, 0)),
        out_shape=jax.ShapeDtypeStruct((G, 1, PH), jnp.float32),
    )(*ins)

    value = sums[:, 0, :].reshape(G * PK, HID).sum(axis=1) * (1.0 / N_AGENTS)
    value = value.reshape(BATCH, 1) + fc2_b.reshape(1, 1)
    return (value, rnn_states)
